# baseline (speedup 1.0000x reference)
import functools

import jax
import jax.numpy as jnp
from jax import lax
from jax.experimental import pallas as pl
from jax.experimental.pallas import tpu as pltpu
from jax.experimental.pallas import tpu_sc as plsc

E = 64
H = 768
I = 384
T = 2048
BT = 64
MAXT = 128
P = MAXT * BT
NC = 2
NS = 16
NW = NC * NS
NEG = -1e30


def _route_body(x_ref, gw_ref, d0_ref, d1_ref, w0_ref, w1_ref,
                chg_ref, fid_ref, erun_ref, rr_ref):
    x = x_ref[...]
    gw = gw_ref[...]
    logits = lax.dot_general(x, gw, (((1,), (1,)), ((), ())),
                             preferred_element_type=jnp.float32)
    iota_e = lax.broadcasted_iota(jnp.int32, (T, E), 1).astype(jnp.float32)
    m0 = jnp.max(logits, axis=1, keepdims=True)
    i0 = jnp.min(jnp.where(logits >= m0, iota_e, jnp.float32(E)),
                 axis=1, keepdims=True)
    sel0 = iota_e == i0
    lm = jnp.where(sel0, NEG, logits)
    m1 = jnp.max(lm, axis=1, keepdims=True)
    i1 = jnp.min(jnp.where(lm >= m1, iota_e, jnp.float32(E)),
                 axis=1, keepdims=True)
    sel1 = iota_e == i1
    w0 = 1.0 / (1.0 + jnp.exp(m1 - m0))
    w1 = 1.0 - w0

    oh0 = sel0.astype(jnp.float32)
    oh1 = sel1.astype(jnp.float32)
    rt = lax.broadcasted_iota(jnp.int32, (T, T), 0)
    ct = lax.broadcasted_iota(jnp.int32, (T, T), 1)
    slt = (rt > ct).astype(jnp.float32)
    cum0 = lax.dot_general(slt, oh0, (((1,), (0,)), ((), ())),
                           preferred_element_type=jnp.float32)
    cum1 = lax.dot_general(slt, oh1, (((1,), (0,)), ((), ())),
                           preferred_element_type=jnp.float32)
    cnt0 = jnp.sum(oh0, axis=0, keepdims=True)
    cnt1 = jnp.sum(oh1, axis=0, keepdims=True)
    cnt = cnt0 + cnt1
    pc = 64.0 * jnp.floor((cnt + 63.0) * (1.0 / 64.0))
    re = lax.broadcasted_iota(jnp.int32, (E, E), 0)
    ce = lax.broadcasted_iota(jnp.int32, (E, E), 1)
    sut = (re < ce).astype(jnp.float32)
    off = lax.dot_general(pc, sut, (((1,), (0,)), ((), ())),
                          preferred_element_type=jnp.float32)
    r0 = jnp.sum(oh0 * cum0, axis=1, keepdims=True)
    r1 = jnp.sum(oh1 * cum1, axis=1, keepdims=True)
    off0 = jnp.sum(oh0 * off, axis=1, keepdims=True)
    off1 = jnp.sum(oh1 * (off + cnt0), axis=1, keepdims=True)
    d0_ref[...] = (off0 + r0).astype(jnp.int32)[:, 0]
    d1_ref[...] = (off1 + r1).astype(jnp.int32)[:, 0]
    w0_ref[...] = w0[:, 0]
    w1_ref[...] = w1[:, 0]
    mt = (lax.broadcasted_iota(jnp.int32, (MAXT, E), 0).astype(jnp.float32)
          * float(BT))
    te_iota = lax.broadcasted_iota(jnp.int32, (MAXT, E), 1)
    temask = (off <= mt) & (pc > 0.0)
    te_col = jnp.max(jnp.where(temask, te_iota, -1), axis=1, keepdims=True)
    te_f = te_col.astype(jnp.float32)
    te_prev = jnp.concatenate(
        [jnp.full((1, 1), -1.0, jnp.float32), te_f[:-1]], axis=0)
    chg = (te_f != te_prev).astype(jnp.float32)
    rm = lax.broadcasted_iota(jnp.int32, (MAXT, MAXT), 0)
    cm = lax.broadcasted_iota(jnp.int32, (MAXT, MAXT), 1)
    lti = (rm >= cm).astype(jnp.float32)
    fid = lax.dot_general(lti, chg, (((1,), (0,)), ((), ())),
                          preferred_element_type=jnp.float32) - 1.0
    run_iota = lax.broadcasted_iota(jnp.int32, (MAXT, MAXT), 1)
    sel_run = (fid == run_iota.astype(jnp.float32)).astype(jnp.float32) * chg
    erun = lax.dot_general(sel_run, te_f, (((0,), (0,)), ((), ())),
                           preferred_element_type=jnp.float32)
    chg_ref[...] = chg.astype(jnp.int32)[:, 0]
    fid_ref[...] = fid.astype(jnp.int32)[:, 0]
    erun_ref[...] = erun.astype(jnp.int32)[:, 0]
    rr_ref[...] = jnp.sum(chg, axis=0).astype(jnp.int32)


def _route(x, gate_w):
    return pl.pallas_call(
        _route_body,
        out_shape=(
            jax.ShapeDtypeStruct((T,), jnp.int32),
            jax.ShapeDtypeStruct((T,), jnp.int32),
            jax.ShapeDtypeStruct((T,), jnp.float32),
            jax.ShapeDtypeStruct((T,), jnp.float32),
            jax.ShapeDtypeStruct((MAXT,), jnp.int32),
            jax.ShapeDtypeStruct((MAXT,), jnp.int32),
            jax.ShapeDtypeStruct((MAXT,), jnp.int32),
            jax.ShapeDtypeStruct((1,), jnp.int32),
        ),
    )(x, gate_w)


def _mesh():
    return plsc.VectorSubcoreMesh(core_axis_name="c", subcore_axis_name="s",
                                  num_cores=NC, num_subcores=NS)


_DSEG = T // NW


def _dispatch_x(d0_hbm, d1_hbm, w0_hbm, w1_hbm, x_hbm,
                xp_hbm, wp_hbm,
                i0_v, i1_v, f0_v, f1_v, rows_v,
                s0, s1, s2, s3, s4):
    wid = lax.axis_index("s") * NC + lax.axis_index("c")
    base = wid * _DSEG
    la = pltpu.async_copy(d0_hbm.at[pl.ds(base, _DSEG)], i0_v, s0)
    lb = pltpu.async_copy(d1_hbm.at[pl.ds(base, _DSEG)], i1_v, s1)
    lc = pltpu.async_copy(w0_hbm.at[pl.ds(base, _DSEG)], f0_v, s2)
    ld = pltpu.async_copy(w1_hbm.at[pl.ds(base, _DSEG)], f1_v, s3)
    lx = pltpu.async_copy(x_hbm.at[pl.ds(base, _DSEG)], rows_v, s4)
    la.wait()
    lx.wait()
    sa = pltpu.async_copy(rows_v, xp_hbm.at[i0_v], s0)
    lb.wait()
    sb = pltpu.async_copy(rows_v, xp_hbm.at[i1_v], s1)
    lc.wait()
    sc = pltpu.async_copy(f0_v, wp_hbm.at[i0_v], s2)
    ld.wait()
    sd = pltpu.async_copy(f1_v, wp_hbm.at[i1_v], s3)
    sa.wait()
    sb.wait()
    sc.wait()
    sd.wait()


def _dispatch(d0, d1, w0, w1, x):
    f = pl.kernel(
        _dispatch_x,
        out_type=(jax.ShapeDtypeStruct((P, H), jnp.float32),
                  jax.ShapeDtypeStruct((P,), jnp.float32)),
        mesh=_mesh(),
        scratch_types=[
            pltpu.VMEM((_DSEG,), jnp.int32),
            pltpu.VMEM((_DSEG,), jnp.int32),
            pltpu.VMEM((_DSEG,), jnp.float32),
            pltpu.VMEM((_DSEG,), jnp.float32),
            pltpu.VMEM((_DSEG, H), jnp.float32),
            pltpu.SemaphoreType.DMA,
            pltpu.SemaphoreType.DMA,
            pltpu.SemaphoreType.DMA,
            pltpu.SemaphoreType.DMA,
            pltpu.SemaphoreType.DMA,
        ],
    )
    return f(d0, d1, w0, w1, x)


_DEPTH = 4


def _ffn_body(chg_ref, fid_ref, erun_ref, rr_ref,
              x_ref, wg_hbm, wu_hbm, wd_hbm, ws_ref, y_ref,
              wg_b, wu_b, wd_b, sg, su, sd):
    m = pl.program_id(0)
    nrun = rr_ref[0]
    fidm = fid_ref[m]
    slot = lax.rem(fidm, _DEPTH)

    def issue(r):
        s = lax.rem(r, _DEPTH)
        e = erun_ref[r]
        pltpu.make_async_copy(wg_hbm.at[e], wg_b.at[s], sg.at[s]).start()
        pltpu.make_async_copy(wu_hbm.at[e], wu_b.at[s], su.at[s]).start()
        pltpu.make_async_copy(wd_hbm.at[e], wd_b.at[s], sd.at[s]).start()

    @pl.when(m == 0)
    def _():
        for d in range(_DEPTH):
            @pl.when(d < nrun)
            def _():
                issue(jnp.int32(d))

    @pl.when((m > 0) & (chg_ref[m] == 1))
    def _():
        r = fidm + (_DEPTH - 1)

        @pl.when(r < nrun)
        def _():
            issue(r)

    @pl.when(chg_ref[m] == 1)
    def _():
        pltpu.make_async_copy(wg_hbm.at[0], wg_b.at[slot], sg.at[slot]).wait()
        pltpu.make_async_copy(wu_hbm.at[0], wu_b.at[slot], su.at[slot]).wait()
        pltpu.make_async_copy(wd_hbm.at[0], wd_b.at[slot], sd.at[slot]).wait()

    xb = x_ref[...].astype(jnp.bfloat16)
    wg = wg_b[slot].astype(jnp.bfloat16)
    wu = wu_b[slot].astype(jnp.bfloat16)
    g = lax.dot_general(xb, wg, (((1,), (1,)), ((), ())),
                        preferred_element_type=jnp.float32)
    u = lax.dot_general(xb, wu, (((1,), (1,)), ((), ())),
                        preferred_element_type=jnp.float32)
    h = g * (1.0 / (1.0 + jnp.exp(-g))) * u
    hb = h.astype(jnp.bfloat16)
    wd = wd_b[slot].astype(jnp.bfloat16)
    y = lax.dot_general(hb, wd, (((1,), (1,)), ((), ())),
                        preferred_element_type=jnp.float32)
    y_ref[...] = y * ws_ref[...]


def _ffn(chg, fid, erun, rr, x_pad, w_gate, w_up, w_down, w_scale):
    grid_spec = pltpu.PrefetchScalarGridSpec(
        num_scalar_prefetch=4,
        grid=(MAXT,),
        in_specs=[
            pl.BlockSpec((BT, H), lambda m, *_: (m, 0)),
            pl.BlockSpec(memory_space=pl.ANY),
            pl.BlockSpec(memory_space=pl.ANY),
            pl.BlockSpec(memory_space=pl.ANY),
            pl.BlockSpec((BT, 1), lambda m, *_: (m, 0)),
        ],
        out_specs=pl.BlockSpec((BT, H), lambda m, *_: (m, 0)),
        scratch_shapes=[
            pltpu.VMEM((_DEPTH, I, H), jnp.float32),
            pltpu.VMEM((_DEPTH, I, H), jnp.float32),
            pltpu.VMEM((_DEPTH, H, I), jnp.float32),
            pltpu.SemaphoreType.DMA((_DEPTH,)),
            pltpu.SemaphoreType.DMA((_DEPTH,)),
            pltpu.SemaphoreType.DMA((_DEPTH,)),
        ],
    )
    return pl.pallas_call(
        _ffn_body,
        grid_spec=grid_spec,
        out_shape=jax.ShapeDtypeStruct((P, H), jnp.float32),
    )(chg, fid, erun, rr, x_pad, w_gate, w_up, w_down, w_scale)


_CSEG = T // NW


def _combine(d0_hbm, d1_hbm, y_hbm, out_hbm, i0_v, i1_v, a_v, b_v, sem,
             sem2):
    wid = lax.axis_index("s") * NC + lax.axis_index("c")
    base = wid * _CSEG
    l0 = pltpu.async_copy(d0_hbm.at[pl.ds(base, _CSEG)], i0_v, sem)
    l1 = pltpu.async_copy(d1_hbm.at[pl.ds(base, _CSEG)], i1_v, sem2)
    l0.wait()
    g0 = pltpu.async_copy(y_hbm.at[i0_v], a_v, sem)
    l1.wait()
    g1 = pltpu.async_copy(y_hbm.at[i1_v], b_v, sem2)
    g0.wait()
    g1.wait()

    def row(r, _):
        def col(j, _):
            s = pl.ds(j * 16, 16)
            a_v[r, s] = a_v[r, s] + b_v[r, s]
            return _
        return lax.fori_loop(0, H // 16, col, _)

    lax.fori_loop(0, _CSEG, row, 0)
    pltpu.sync_copy(a_v, out_hbm.at[pl.ds(base, _CSEG)])


def _combine_call(d0, d1, y_pad):
    f = pl.kernel(
        _combine,
        out_type=jax.ShapeDtypeStruct((T, H), jnp.float32),
        mesh=_mesh(),
        scratch_types=[
            pltpu.VMEM((_CSEG,), jnp.int32),
            pltpu.VMEM((_CSEG,), jnp.int32),
            pltpu.VMEM((_CSEG, H), jnp.float32),
            pltpu.VMEM((_CSEG, H), jnp.float32),
            pltpu.SemaphoreType.DMA,
            pltpu.SemaphoreType.DMA,
        ],
    )
    return f(d0, d1, y_pad)


def kernel(hidden_states, gate_w, w_gate, w_up, w_down):
    d0, d1, w0, w1, chg, fid, erun, rr = _route(hidden_states, gate_w)
    x_pad, w_pad = _dispatch(d0, d1, w0, w1, hidden_states)
    y_pad = _ffn(chg, fid, erun, rr, x_pad, w_gate, w_up, w_down,
                 w_pad.reshape(P, 1))
    return _combine_call(d0, d1, y_pad)


# speedup vs baseline: 1.0338x; 1.0338x over previous
import functools

import jax
import jax.numpy as jnp
from jax import lax
from jax.experimental import pallas as pl
from jax.experimental.pallas import tpu as pltpu
from jax.experimental.pallas import tpu_sc as plsc

E = 64
H = 768
I = 384
T = 2048
BT = 64
MAXT = 128
P = MAXT * BT
NC = 2
NS = 16
NW = NC * NS
NEG = -1e30


def _route_body(x_ref, gw_ref, d0_ref, d1_ref, w0_ref, w1_ref,
                chg_ref, fid_ref, erun_ref, rr_ref):
    x = x_ref[...]
    gw = gw_ref[...]
    logits = lax.dot_general(x, gw, (((1,), (1,)), ((), ())),
                             preferred_element_type=jnp.float32)
    iota_e = lax.broadcasted_iota(jnp.int32, (T, E), 1).astype(jnp.float32)
    m0 = jnp.max(logits, axis=1, keepdims=True)
    i0 = jnp.min(jnp.where(logits >= m0, iota_e, jnp.float32(E)),
                 axis=1, keepdims=True)
    sel0 = iota_e == i0
    lm = jnp.where(sel0, NEG, logits)
    m1 = jnp.max(lm, axis=1, keepdims=True)
    i1 = jnp.min(jnp.where(lm >= m1, iota_e, jnp.float32(E)),
                 axis=1, keepdims=True)
    sel1 = iota_e == i1
    w0 = 1.0 / (1.0 + jnp.exp(m1 - m0))
    w1 = 1.0 - w0

    oh0 = sel0.astype(jnp.float32)
    oh1 = sel1.astype(jnp.float32)
    rt = lax.broadcasted_iota(jnp.int32, (T, T), 0)
    ct = lax.broadcasted_iota(jnp.int32, (T, T), 1)
    slt = (rt > ct).astype(jnp.float32)
    cum0 = lax.dot_general(slt, oh0, (((1,), (0,)), ((), ())),
                           preferred_element_type=jnp.float32)
    cum1 = lax.dot_general(slt, oh1, (((1,), (0,)), ((), ())),
                           preferred_element_type=jnp.float32)
    cnt0 = jnp.sum(oh0, axis=0, keepdims=True)
    cnt1 = jnp.sum(oh1, axis=0, keepdims=True)
    cnt = cnt0 + cnt1
    pc = 64.0 * jnp.floor((cnt + 63.0) * (1.0 / 64.0))
    re = lax.broadcasted_iota(jnp.int32, (E, E), 0)
    ce = lax.broadcasted_iota(jnp.int32, (E, E), 1)
    sut = (re < ce).astype(jnp.float32)
    off = lax.dot_general(pc, sut, (((1,), (0,)), ((), ())),
                          preferred_element_type=jnp.float32)
    r0 = jnp.sum(oh0 * cum0, axis=1, keepdims=True)
    r1 = jnp.sum(oh1 * cum1, axis=1, keepdims=True)
    off0 = jnp.sum(oh0 * off, axis=1, keepdims=True)
    off1 = jnp.sum(oh1 * (off + cnt0), axis=1, keepdims=True)
    d0_ref[...] = (off0 + r0).astype(jnp.int32)[:, 0]
    d1_ref[...] = (off1 + r1).astype(jnp.int32)[:, 0]
    w0_ref[...] = w0[:, 0]
    w1_ref[...] = w1[:, 0]
    mt = (lax.broadcasted_iota(jnp.int32, (MAXT, E), 0).astype(jnp.float32)
          * float(BT))
    te_iota = lax.broadcasted_iota(jnp.int32, (MAXT, E), 1)
    temask = (off <= mt) & (pc > 0.0)
    te_col = jnp.max(jnp.where(temask, te_iota, -1), axis=1, keepdims=True)
    te_f = te_col.astype(jnp.float32)
    te_prev = jnp.concatenate(
        [jnp.full((1, 1), -1.0, jnp.float32), te_f[:-1]], axis=0)
    chg = (te_f != te_prev).astype(jnp.float32)
    rm = lax.broadcasted_iota(jnp.int32, (MAXT, MAXT), 0)
    cm = lax.broadcasted_iota(jnp.int32, (MAXT, MAXT), 1)
    lti = (rm >= cm).astype(jnp.float32)
    fid = lax.dot_general(lti, chg, (((1,), (0,)), ((), ())),
                          preferred_element_type=jnp.float32) - 1.0
    run_iota = lax.broadcasted_iota(jnp.int32, (MAXT, MAXT), 1)
    sel_run = (fid == run_iota.astype(jnp.float32)).astype(jnp.float32) * chg
    erun = lax.dot_general(sel_run, te_f, (((0,), (0,)), ((), ())),
                           preferred_element_type=jnp.float32)
    chg_ref[...] = chg.astype(jnp.int32)[:, 0]
    fid_ref[...] = fid.astype(jnp.int32)[:, 0]
    erun_ref[...] = erun.astype(jnp.int32)[:, 0]
    rr_ref[...] = jnp.sum(chg, axis=0).astype(jnp.int32)


def _route(x, gate_w):
    return pl.pallas_call(
        _route_body,
        out_shape=(
            jax.ShapeDtypeStruct((T,), jnp.int32),
            jax.ShapeDtypeStruct((T,), jnp.int32),
            jax.ShapeDtypeStruct((T,), jnp.float32),
            jax.ShapeDtypeStruct((T,), jnp.float32),
            jax.ShapeDtypeStruct((MAXT,), jnp.int32),
            jax.ShapeDtypeStruct((MAXT,), jnp.int32),
            jax.ShapeDtypeStruct((MAXT,), jnp.int32),
            jax.ShapeDtypeStruct((1,), jnp.int32),
        ),
    )(x, gate_w)


def _mesh():
    return plsc.VectorSubcoreMesh(core_axis_name="c", subcore_axis_name="s",
                                  num_cores=NC, num_subcores=NS)


_DSEG = T // NW


def _dispatch_x(d0_hbm, d1_hbm, w0_hbm, w1_hbm, x_hbm,
                xp_hbm, wp_hbm,
                i0_v, i1_v, f0_v, f1_v, rows_v,
                s0, s1, s2, s3, s4):
    wid = lax.axis_index("s") * NC + lax.axis_index("c")
    base = wid * _DSEG
    la = pltpu.async_copy(d0_hbm.at[pl.ds(base, _DSEG)], i0_v, s0)
    lb = pltpu.async_copy(d1_hbm.at[pl.ds(base, _DSEG)], i1_v, s1)
    lc = pltpu.async_copy(w0_hbm.at[pl.ds(base, _DSEG)], f0_v, s2)
    ld = pltpu.async_copy(w1_hbm.at[pl.ds(base, _DSEG)], f1_v, s3)
    lx = pltpu.async_copy(x_hbm.at[pl.ds(base, _DSEG)], rows_v, s4)
    la.wait()
    lx.wait()
    sa = pltpu.async_copy(rows_v, xp_hbm.at[i0_v], s0)
    lb.wait()
    sb = pltpu.async_copy(rows_v, xp_hbm.at[i1_v], s1)
    lc.wait()
    sc = pltpu.async_copy(f0_v, wp_hbm.at[i0_v], s2)
    ld.wait()
    sd = pltpu.async_copy(f1_v, wp_hbm.at[i1_v], s3)
    sa.wait()
    sb.wait()
    sc.wait()
    sd.wait()


def _dispatch(d0, d1, w0, w1, x):
    f = pl.kernel(
        _dispatch_x,
        out_type=(jax.ShapeDtypeStruct((P, H), jnp.float32),
                  jax.ShapeDtypeStruct((P,), jnp.float32)),
        mesh=_mesh(),
        scratch_types=[
            pltpu.VMEM((_DSEG,), jnp.int32),
            pltpu.VMEM((_DSEG,), jnp.int32),
            pltpu.VMEM((_DSEG,), jnp.float32),
            pltpu.VMEM((_DSEG,), jnp.float32),
            pltpu.VMEM((_DSEG, H), jnp.float32),
            pltpu.SemaphoreType.DMA,
            pltpu.SemaphoreType.DMA,
            pltpu.SemaphoreType.DMA,
            pltpu.SemaphoreType.DMA,
            pltpu.SemaphoreType.DMA,
        ],
    )
    return f(d0, d1, w0, w1, x)


_DEPTH = 6


def _ffn_body(chg_ref, fid_ref, erun_ref, rr_ref,
              x_ref, wg_hbm, wu_hbm, wd_hbm, ws_ref, y_ref,
              wg_b, wu_b, wd_b, sg, su, sd):
    m = pl.program_id(0)
    nrun = rr_ref[0]
    fidm = fid_ref[m]
    slot = lax.rem(fidm, _DEPTH)

    def issue(r):
        s = lax.rem(r, _DEPTH)
        e = erun_ref[r]
        pltpu.make_async_copy(wg_hbm.at[e], wg_b.at[s], sg.at[s]).start()
        pltpu.make_async_copy(wu_hbm.at[e], wu_b.at[s], su.at[s]).start()
        pltpu.make_async_copy(wd_hbm.at[e], wd_b.at[s], sd.at[s]).start()

    @pl.when(m == 0)
    def _():
        for d in range(_DEPTH):
            @pl.when(d < nrun)
            def _():
                issue(jnp.int32(d))

    @pl.when((m > 0) & (chg_ref[m] == 1))
    def _():
        r = fidm + (_DEPTH - 1)

        @pl.when(r < nrun)
        def _():
            issue(r)

    @pl.when(chg_ref[m] == 1)
    def _():
        pltpu.make_async_copy(wg_hbm.at[0], wg_b.at[slot], sg.at[slot]).wait()
        pltpu.make_async_copy(wu_hbm.at[0], wu_b.at[slot], su.at[slot]).wait()
        pltpu.make_async_copy(wd_hbm.at[0], wd_b.at[slot], sd.at[slot]).wait()

    xb = x_ref[...].astype(jnp.bfloat16)
    wg = wg_b[slot].astype(jnp.bfloat16)
    wu = wu_b[slot].astype(jnp.bfloat16)
    g = lax.dot_general(xb, wg, (((1,), (1,)), ((), ())),
                        preferred_element_type=jnp.float32)
    u = lax.dot_general(xb, wu, (((1,), (1,)), ((), ())),
                        preferred_element_type=jnp.float32)
    h = g * (1.0 / (1.0 + jnp.exp(-g))) * u
    hb = h.astype(jnp.bfloat16)
    wd = wd_b[slot].astype(jnp.bfloat16)
    y = lax.dot_general(hb, wd, (((1,), (1,)), ((), ())),
                        preferred_element_type=jnp.float32)
    y_ref[...] = y * ws_ref[...]


def _ffn(chg, fid, erun, rr, x_pad, w_gate, w_up, w_down, w_scale):
    grid_spec = pltpu.PrefetchScalarGridSpec(
        num_scalar_prefetch=4,
        grid=(MAXT,),
        in_specs=[
            pl.BlockSpec((BT, H), lambda m, *_: (m, 0)),
            pl.BlockSpec(memory_space=pl.ANY),
            pl.BlockSpec(memory_space=pl.ANY),
            pl.BlockSpec(memory_space=pl.ANY),
            pl.BlockSpec((BT, 1), lambda m, *_: (m, 0)),
        ],
        out_specs=pl.BlockSpec((BT, H), lambda m, *_: (m, 0)),
        scratch_shapes=[
            pltpu.VMEM((_DEPTH, I, H), jnp.float32),
            pltpu.VMEM((_DEPTH, I, H), jnp.float32),
            pltpu.VMEM((_DEPTH, H, I), jnp.float32),
            pltpu.SemaphoreType.DMA((_DEPTH,)),
            pltpu.SemaphoreType.DMA((_DEPTH,)),
            pltpu.SemaphoreType.DMA((_DEPTH,)),
        ],
    )
    return pl.pallas_call(
        _ffn_body,
        grid_spec=grid_spec,
        out_shape=jax.ShapeDtypeStruct((P, H), jnp.float32),
    )(chg, fid, erun, rr, x_pad, w_gate, w_up, w_down, w_scale)


_CSEG = T // NW


def _combine(d0_hbm, d1_hbm, y_hbm, out_hbm, i0_v, i1_v, a_v, b_v, sem,
             sem2):
    wid = lax.axis_index("s") * NC + lax.axis_index("c")
    base = wid * _CSEG
    l0 = pltpu.async_copy(d0_hbm.at[pl.ds(base, _CSEG)], i0_v, sem)
    l1 = pltpu.async_copy(d1_hbm.at[pl.ds(base, _CSEG)], i1_v, sem2)
    l0.wait()
    g0 = pltpu.async_copy(y_hbm.at[i0_v], a_v, sem)
    l1.wait()
    g1 = pltpu.async_copy(y_hbm.at[i1_v], b_v, sem2)
    g0.wait()
    g1.wait()

    def row(r, _):
        for j in range(H // 16):
            s = pl.ds(j * 16, 16)
            a_v[r, s] = a_v[r, s] + b_v[r, s]
        return _

    lax.fori_loop(0, _CSEG, row, 0)
    pltpu.sync_copy(a_v, out_hbm.at[pl.ds(base, _CSEG)])


def _combine_call(d0, d1, y_pad):
    f = pl.kernel(
        _combine,
        out_type=jax.ShapeDtypeStruct((T, H), jnp.float32),
        mesh=_mesh(),
        scratch_types=[
            pltpu.VMEM((_CSEG,), jnp.int32),
            pltpu.VMEM((_CSEG,), jnp.int32),
            pltpu.VMEM((_CSEG, H), jnp.float32),
            pltpu.VMEM((_CSEG, H), jnp.float32),
            pltpu.SemaphoreType.DMA,
            pltpu.SemaphoreType.DMA,
        ],
    )
    return f(d0, d1, y_pad)


def kernel(hidden_states, gate_w, w_gate, w_up, w_down):
    d0, d1, w0, w1, chg, fid, erun, rr = _route(hidden_states, gate_w)
    x_pad, w_pad = _dispatch(d0, d1, w0, w1, hidden_states)
    y_pad = _ffn(chg, fid, erun, rr, x_pad, w_gate, w_up, w_down,
                 w_pad.reshape(P, 1))
    return _combine_call(d0, d1, y_pad)


# speedup vs baseline: 1.1348x; 1.0977x over previous
import functools

import jax
import jax.numpy as jnp
from jax import lax
from jax.experimental import pallas as pl
from jax.experimental.pallas import tpu as pltpu
from jax.experimental.pallas import tpu_sc as plsc

E = 64
H = 768
I = 384
T = 2048
BT = 64
MAXT = 128
P = MAXT * BT
NC = 2
NS = 16
NW = NC * NS
NEG = -1e30


def _route_body(x_ref, gw_ref, d0_ref, d1_ref, w0_ref, w1_ref,
                chg_ref, fid_ref, erun_ref, rr_ref, ntl_ref):
    x = x_ref[...]
    gw = gw_ref[...]
    logits = lax.dot_general(x, gw, (((1,), (1,)), ((), ())),
                             preferred_element_type=jnp.float32)
    iota_e = lax.broadcasted_iota(jnp.int32, (T, E), 1).astype(jnp.float32)
    m0 = jnp.max(logits, axis=1, keepdims=True)
    i0 = jnp.min(jnp.where(logits >= m0, iota_e, jnp.float32(E)),
                 axis=1, keepdims=True)
    sel0 = iota_e == i0
    lm = jnp.where(sel0, NEG, logits)
    m1 = jnp.max(lm, axis=1, keepdims=True)
    i1 = jnp.min(jnp.where(lm >= m1, iota_e, jnp.float32(E)),
                 axis=1, keepdims=True)
    sel1 = iota_e == i1
    w0 = 1.0 / (1.0 + jnp.exp(m1 - m0))
    w1 = 1.0 - w0

    oh0 = sel0.astype(jnp.float32)
    oh1 = sel1.astype(jnp.float32)
    rt = lax.broadcasted_iota(jnp.int32, (T, T), 0)
    ct = lax.broadcasted_iota(jnp.int32, (T, T), 1)
    slt = (rt > ct).astype(jnp.float32)
    cum0 = lax.dot_general(slt, oh0, (((1,), (0,)), ((), ())),
                           preferred_element_type=jnp.float32)
    cum1 = lax.dot_general(slt, oh1, (((1,), (0,)), ((), ())),
                           preferred_element_type=jnp.float32)
    cnt0 = jnp.sum(oh0, axis=0, keepdims=True)
    cnt1 = jnp.sum(oh1, axis=0, keepdims=True)
    cnt = cnt0 + cnt1
    pc = 64.0 * jnp.floor((cnt + 63.0) * (1.0 / 64.0))
    re = lax.broadcasted_iota(jnp.int32, (E, E), 0)
    ce = lax.broadcasted_iota(jnp.int32, (E, E), 1)
    sut = (re < ce).astype(jnp.float32)
    off = lax.dot_general(pc, sut, (((1,), (0,)), ((), ())),
                          preferred_element_type=jnp.float32)
    r0 = jnp.sum(oh0 * cum0, axis=1, keepdims=True)
    r1 = jnp.sum(oh1 * cum1, axis=1, keepdims=True)
    off0 = jnp.sum(oh0 * off, axis=1, keepdims=True)
    off1 = jnp.sum(oh1 * (off + cnt0), axis=1, keepdims=True)
    d0_ref[...] = (off0 + r0).astype(jnp.int32)[:, 0]
    d1_ref[...] = (off1 + r1).astype(jnp.int32)[:, 0]
    w0_ref[...] = w0[:, 0]
    w1_ref[...] = w1[:, 0]
    mt = (lax.broadcasted_iota(jnp.int32, (MAXT, E), 0).astype(jnp.float32)
          * float(BT))
    te_iota = lax.broadcasted_iota(jnp.int32, (MAXT, E), 1)
    temask = (off <= mt) & (pc > 0.0)
    te_col = jnp.max(jnp.where(temask, te_iota, -1), axis=1, keepdims=True)
    te_f = te_col.astype(jnp.float32)
    te_prev = jnp.concatenate(
        [jnp.full((1, 1), -1.0, jnp.float32), te_f[:-1]], axis=0)
    chg = (te_f != te_prev).astype(jnp.float32)
    rm = lax.broadcasted_iota(jnp.int32, (MAXT, MAXT), 0)
    cm = lax.broadcasted_iota(jnp.int32, (MAXT, MAXT), 1)
    lti = (rm >= cm).astype(jnp.float32)
    fid = lax.dot_general(lti, chg, (((1,), (0,)), ((), ())),
                          preferred_element_type=jnp.float32) - 1.0
    run_iota = lax.broadcasted_iota(jnp.int32, (MAXT, MAXT), 1)
    sel_run = (fid == run_iota.astype(jnp.float32)).astype(jnp.float32) * chg
    erun = lax.dot_general(sel_run, te_f, (((0,), (0,)), ((), ())),
                           preferred_element_type=jnp.float32)
    chg_ref[...] = chg.astype(jnp.int32)[:, 0]
    fid_ref[...] = fid.astype(jnp.int32)[:, 0]
    erun_ref[...] = erun.astype(jnp.int32)[:, 0]
    rr_ref[...] = jnp.sum(chg, axis=0).astype(jnp.int32)
    ntl_ref[...] = (jnp.sum(pc, axis=1) * (1.0 / BT)).astype(jnp.int32)


def _route(x, gate_w):
    return pl.pallas_call(
        _route_body,
        out_shape=(
            jax.ShapeDtypeStruct((T,), jnp.int32),
            jax.ShapeDtypeStruct((T,), jnp.int32),
            jax.ShapeDtypeStruct((T,), jnp.float32),
            jax.ShapeDtypeStruct((T,), jnp.float32),
            jax.ShapeDtypeStruct((MAXT,), jnp.int32),
            jax.ShapeDtypeStruct((MAXT,), jnp.int32),
            jax.ShapeDtypeStruct((MAXT,), jnp.int32),
            jax.ShapeDtypeStruct((1,), jnp.int32),
            jax.ShapeDtypeStruct((1,), jnp.int32),
        ),
    )(x, gate_w)


def _mesh():
    return plsc.VectorSubcoreMesh(core_axis_name="c", subcore_axis_name="s",
                                  num_cores=NC, num_subcores=NS)


_DSEG = T // NW


def _dispatch_x(d0_hbm, d1_hbm, w0_hbm, w1_hbm, x_hbm,
                xp_hbm, wp_hbm,
                i0_v, i1_v, f0_v, f1_v, rows_v,
                s0, s1, s2, s3, s4):
    wid = lax.axis_index("s") * NC + lax.axis_index("c")
    base = wid * _DSEG
    la = pltpu.async_copy(d0_hbm.at[pl.ds(base, _DSEG)], i0_v, s0)
    lb = pltpu.async_copy(d1_hbm.at[pl.ds(base, _DSEG)], i1_v, s1)
    lc = pltpu.async_copy(w0_hbm.at[pl.ds(base, _DSEG)], f0_v, s2)
    ld = pltpu.async_copy(w1_hbm.at[pl.ds(base, _DSEG)], f1_v, s3)
    lx = pltpu.async_copy(x_hbm.at[pl.ds(base, _DSEG)], rows_v, s4)
    la.wait()
    lx.wait()
    sa = pltpu.async_copy(rows_v, xp_hbm.at[i0_v], s0)
    lb.wait()
    sb = pltpu.async_copy(rows_v, xp_hbm.at[i1_v], s1)
    lc.wait()
    sc = pltpu.async_copy(f0_v, wp_hbm.at[i0_v], s2)
    ld.wait()
    sd = pltpu.async_copy(f1_v, wp_hbm.at[i1_v], s3)
    sa.wait()
    sb.wait()
    sc.wait()
    sd.wait()


def _dispatch(d0, d1, w0, w1, x):
    f = pl.kernel(
        _dispatch_x,
        out_type=(jax.ShapeDtypeStruct((P, H), jnp.float32),
                  jax.ShapeDtypeStruct((P,), jnp.float32)),
        mesh=_mesh(),
        scratch_types=[
            pltpu.VMEM((_DSEG,), jnp.int32),
            pltpu.VMEM((_DSEG,), jnp.int32),
            pltpu.VMEM((_DSEG,), jnp.float32),
            pltpu.VMEM((_DSEG,), jnp.float32),
            pltpu.VMEM((_DSEG, H), jnp.float32),
            pltpu.SemaphoreType.DMA,
            pltpu.SemaphoreType.DMA,
            pltpu.SemaphoreType.DMA,
            pltpu.SemaphoreType.DMA,
            pltpu.SemaphoreType.DMA,
        ],
    )
    return f(d0, d1, w0, w1, x)


_DEPTH = 6


def _ffn_body(chg_ref, fid_ref, erun_ref, rr_ref, ntl_ref,
              x_ref, wg_hbm, wu_hbm, wd_hbm, ws_ref, y_ref,
              wg_b, wu_b, wd_b, sg, su, sd):
    m = pl.program_id(0)
    nrun = rr_ref[0]
    fidm = fid_ref[m]
    slot = lax.rem(fidm, _DEPTH)

    def issue(r):
        s = lax.rem(r, _DEPTH)
        e = erun_ref[r]
        pltpu.make_async_copy(wg_hbm.at[e], wg_b.at[s], sg.at[s]).start()
        pltpu.make_async_copy(wu_hbm.at[e], wu_b.at[s], su.at[s]).start()
        pltpu.make_async_copy(wd_hbm.at[e], wd_b.at[s], sd.at[s]).start()

    @pl.when(m == 0)
    def _():
        for d in range(_DEPTH):
            @pl.when(d < nrun)
            def _():
                issue(jnp.int32(d))

    @pl.when((m > 0) & (chg_ref[m] == 1))
    def _():
        r = fidm + (_DEPTH - 1)

        @pl.when(r < nrun)
        def _():
            issue(r)

    @pl.when(chg_ref[m] == 1)
    def _():
        pltpu.make_async_copy(wg_hbm.at[0], wg_b.at[slot], sg.at[slot]).wait()
        pltpu.make_async_copy(wu_hbm.at[0], wu_b.at[slot], su.at[slot]).wait()
        pltpu.make_async_copy(wd_hbm.at[0], wd_b.at[slot], sd.at[slot]).wait()

    @pl.when(m < ntl_ref[0])
    def _():
        xb = x_ref[...].astype(jnp.bfloat16)
        wg = wg_b[slot].astype(jnp.bfloat16)
        wu = wu_b[slot].astype(jnp.bfloat16)
        g = lax.dot_general(xb, wg, (((1,), (1,)), ((), ())),
                            preferred_element_type=jnp.float32)
        u = lax.dot_general(xb, wu, (((1,), (1,)), ((), ())),
                            preferred_element_type=jnp.float32)
        h = g * (1.0 / (1.0 + jnp.exp(-g))) * u
        hb = h.astype(jnp.bfloat16)
        wd = wd_b[slot].astype(jnp.bfloat16)
        y = lax.dot_general(hb, wd, (((1,), (1,)), ((), ())),
                            preferred_element_type=jnp.float32)
        y_ref[...] = y * ws_ref[...]


def _ffn(chg, fid, erun, rr, ntl, x_pad, w_gate, w_up, w_down, w_scale):
    def _xmap(m, chg, fid, erun, rr, ntl):
        return (jnp.minimum(m, ntl[0] - 1), 0)

    grid_spec = pltpu.PrefetchScalarGridSpec(
        num_scalar_prefetch=5,
        grid=(MAXT,),
        in_specs=[
            pl.BlockSpec((BT, H), _xmap),
            pl.BlockSpec(memory_space=pl.ANY),
            pl.BlockSpec(memory_space=pl.ANY),
            pl.BlockSpec(memory_space=pl.ANY),
            pl.BlockSpec((BT, 1), _xmap),
        ],
        out_specs=pl.BlockSpec((BT, H), lambda m, *_: (m, 0)),
        scratch_shapes=[
            pltpu.VMEM((_DEPTH, I, H), jnp.float32),
            pltpu.VMEM((_DEPTH, I, H), jnp.float32),
            pltpu.VMEM((_DEPTH, H, I), jnp.float32),
            pltpu.SemaphoreType.DMA((_DEPTH,)),
            pltpu.SemaphoreType.DMA((_DEPTH,)),
            pltpu.SemaphoreType.DMA((_DEPTH,)),
        ],
    )
    return pl.pallas_call(
        _ffn_body,
        grid_spec=grid_spec,
        out_shape=jax.ShapeDtypeStruct((P, H), jnp.float32),
    )(chg, fid, erun, rr, ntl, x_pad, w_gate, w_up, w_down, w_scale)


_CSEG = T // NW


def _combine(d0_hbm, d1_hbm, y_hbm, out_hbm, i0_v, i1_v, a_v, b_v, sem,
             sem2):
    wid = lax.axis_index("s") * NC + lax.axis_index("c")
    base = wid * _CSEG
    l0 = pltpu.async_copy(d0_hbm.at[pl.ds(base, _CSEG)], i0_v, sem)
    l1 = pltpu.async_copy(d1_hbm.at[pl.ds(base, _CSEG)], i1_v, sem2)
    l0.wait()
    g0 = pltpu.async_copy(y_hbm.at[i0_v], a_v, sem)
    l1.wait()
    g1 = pltpu.async_copy(y_hbm.at[i1_v], b_v, sem2)
    g0.wait()
    g1.wait()

    def row(r, _):
        for j in range(H // 16):
            s = pl.ds(j * 16, 16)
            a_v[r, s] = a_v[r, s] + b_v[r, s]
        return _

    lax.fori_loop(0, _CSEG, row, 0)
    pltpu.sync_copy(a_v, out_hbm.at[pl.ds(base, _CSEG)])


def _combine_call(d0, d1, y_pad):
    f = pl.kernel(
        _combine,
        out_type=jax.ShapeDtypeStruct((T, H), jnp.float32),
        mesh=_mesh(),
        scratch_types=[
            pltpu.VMEM((_CSEG,), jnp.int32),
            pltpu.VMEM((_CSEG,), jnp.int32),
            pltpu.VMEM((_CSEG, H), jnp.float32),
            pltpu.VMEM((_CSEG, H), jnp.float32),
            pltpu.SemaphoreType.DMA,
            pltpu.SemaphoreType.DMA,
        ],
    )
    return f(d0, d1, y_pad)


def kernel(hidden_states, gate_w, w_gate, w_up, w_down):
    d0, d1, w0, w1, chg, fid, erun, rr, ntl = _route(hidden_states, gate_w)
    x_pad, w_pad = _dispatch(d0, d1, w0, w1, hidden_states)
    y_pad = _ffn(chg, fid, erun, rr, ntl, x_pad, w_gate, w_up, w_down,
                 w_pad.reshape(P, 1))
    return _combine_call(d0, d1, y_pad)


# speedup vs baseline: 1.1369x; 1.0018x over previous
import jax
import jax.numpy as jnp
from jax import lax
from jax.experimental import pallas as pl
from jax.experimental.pallas import tpu as pltpu
from jax.experimental.pallas import tpu_sc as plsc

E = 64
H = 768
I = 384
T = 2048
BT = 64
MAXT = 128
P = MAXT * BT
NC = 2
NS = 16
NW = NC * NS
NEG = -1e30


def _route_body(x_ref, gw_ref, d0_ref, d1_ref, w0_ref, w1_ref,
                chg_ref, fid_ref, erun_ref, rr_ref, ntl_ref):
    x = x_ref[...]
    gw = gw_ref[...]
    logits = lax.dot_general(x, gw, (((1,), (1,)), ((), ())),
                             preferred_element_type=jnp.float32)
    iota_e = lax.broadcasted_iota(jnp.int32, (T, E), 1).astype(jnp.float32)
    m0 = jnp.max(logits, axis=1, keepdims=True)
    i0 = jnp.min(jnp.where(logits >= m0, iota_e, jnp.float32(E)),
                 axis=1, keepdims=True)
    sel0 = iota_e == i0
    lm = jnp.where(sel0, NEG, logits)
    m1 = jnp.max(lm, axis=1, keepdims=True)
    i1 = jnp.min(jnp.where(lm >= m1, iota_e, jnp.float32(E)),
                 axis=1, keepdims=True)
    sel1 = iota_e == i1
    w0 = 1.0 / (1.0 + jnp.exp(m1 - m0))
    w1 = 1.0 - w0

    oh0 = sel0.astype(jnp.float32)
    oh1 = sel1.astype(jnp.float32)
    rt = lax.broadcasted_iota(jnp.int32, (T, T), 0)
    ct = lax.broadcasted_iota(jnp.int32, (T, T), 1)
    slt = (rt > ct).astype(jnp.float32)
    cum0 = lax.dot_general(slt, oh0, (((1,), (0,)), ((), ())),
                           preferred_element_type=jnp.float32)
    cum1 = lax.dot_general(slt, oh1, (((1,), (0,)), ((), ())),
                           preferred_element_type=jnp.float32)
    cnt0 = jnp.sum(oh0, axis=0, keepdims=True)
    cnt1 = jnp.sum(oh1, axis=0, keepdims=True)
    cnt = cnt0 + cnt1
    pc = 64.0 * jnp.floor((cnt + 63.0) * (1.0 / 64.0))
    re = lax.broadcasted_iota(jnp.int32, (E, E), 0)
    ce = lax.broadcasted_iota(jnp.int32, (E, E), 1)
    sut = (re < ce).astype(jnp.float32)
    off = lax.dot_general(pc, sut, (((1,), (0,)), ((), ())),
                          preferred_element_type=jnp.float32)
    r0 = jnp.sum(oh0 * cum0, axis=1, keepdims=True)
    r1 = jnp.sum(oh1 * cum1, axis=1, keepdims=True)
    off0 = jnp.sum(oh0 * off, axis=1, keepdims=True)
    off1 = jnp.sum(oh1 * (off + cnt0), axis=1, keepdims=True)
    d0_ref[...] = (off0 + r0).astype(jnp.int32)[:, 0]
    d1_ref[...] = (off1 + r1).astype(jnp.int32)[:, 0]
    w0_ref[...] = w0[:, 0]
    w1_ref[...] = w1[:, 0]
    mt = (lax.broadcasted_iota(jnp.int32, (MAXT, E), 0).astype(jnp.float32)
          * float(BT))
    te_iota = lax.broadcasted_iota(jnp.int32, (MAXT, E), 1)
    temask = (off <= mt) & (pc > 0.0)
    te_col = jnp.max(jnp.where(temask, te_iota, -1), axis=1, keepdims=True)
    te_f = te_col.astype(jnp.float32)
    te_prev = jnp.concatenate(
        [jnp.full((1, 1), -1.0, jnp.float32), te_f[:-1]], axis=0)
    chg = (te_f != te_prev).astype(jnp.float32)
    rm = lax.broadcasted_iota(jnp.int32, (MAXT, MAXT), 0)
    cm = lax.broadcasted_iota(jnp.int32, (MAXT, MAXT), 1)
    lti = (rm >= cm).astype(jnp.float32)
    fid = lax.dot_general(lti, chg, (((1,), (0,)), ((), ())),
                          preferred_element_type=jnp.float32) - 1.0
    run_iota = lax.broadcasted_iota(jnp.int32, (MAXT, MAXT), 1)
    sel_run = (fid == run_iota.astype(jnp.float32)).astype(jnp.float32) * chg
    erun = lax.dot_general(sel_run, te_f, (((0,), (0,)), ((), ())),
                           preferred_element_type=jnp.float32)
    chg_ref[...] = chg.astype(jnp.int32)[:, 0]
    fid_ref[...] = fid.astype(jnp.int32)[:, 0]
    erun_ref[...] = erun.astype(jnp.int32)[:, 0]
    rr_ref[...] = jnp.sum(chg, axis=0).astype(jnp.int32)
    ntl_ref[...] = (jnp.sum(pc, axis=1) * (1.0 / BT)).astype(jnp.int32)


def _route(x, gate_w):
    return pl.pallas_call(
        _route_body,
        out_shape=(
            jax.ShapeDtypeStruct((T,), jnp.int32),
            jax.ShapeDtypeStruct((T,), jnp.int32),
            jax.ShapeDtypeStruct((T,), jnp.float32),
            jax.ShapeDtypeStruct((T,), jnp.float32),
            jax.ShapeDtypeStruct((MAXT,), jnp.int32),
            jax.ShapeDtypeStruct((MAXT,), jnp.int32),
            jax.ShapeDtypeStruct((MAXT,), jnp.int32),
            jax.ShapeDtypeStruct((1,), jnp.int32),
            jax.ShapeDtypeStruct((1,), jnp.int32),
        ),
    )(x, gate_w)


def _mesh():
    return plsc.VectorSubcoreMesh(core_axis_name="c", subcore_axis_name="s",
                                  num_cores=NC, num_subcores=NS)


_DSEG = T // NW


def _dispatch_x(d0_hbm, d1_hbm, w0_hbm, w1_hbm, x_hbm,
                xp_hbm, wp_hbm,
                i0_v, i1_v, f0_v, f1_v, rows_v,
                s0, s1, s2, s3, s4):
    wid = lax.axis_index("s") * NC + lax.axis_index("c")
    base = wid * _DSEG
    la = pltpu.async_copy(d0_hbm.at[pl.ds(base, _DSEG)], i0_v, s0)
    lb = pltpu.async_copy(d1_hbm.at[pl.ds(base, _DSEG)], i1_v, s1)
    lc = pltpu.async_copy(w0_hbm.at[pl.ds(base, _DSEG)], f0_v, s2)
    ld = pltpu.async_copy(w1_hbm.at[pl.ds(base, _DSEG)], f1_v, s3)
    lx = pltpu.async_copy(x_hbm.at[pl.ds(base, _DSEG)], rows_v, s4)
    la.wait()
    lx.wait()
    sa = pltpu.async_copy(rows_v, xp_hbm.at[i0_v], s0)
    lb.wait()
    sb = pltpu.async_copy(rows_v, xp_hbm.at[i1_v], s1)
    lc.wait()
    sc = pltpu.async_copy(f0_v, wp_hbm.at[i0_v], s2)
    ld.wait()
    sd = pltpu.async_copy(f1_v, wp_hbm.at[i1_v], s3)
    sa.wait()
    sb.wait()
    sc.wait()
    sd.wait()


def _dispatch(d0, d1, w0, w1, x):
    f = pl.kernel(
        _dispatch_x,
        out_type=(jax.ShapeDtypeStruct((P, H), jnp.float32),
                  jax.ShapeDtypeStruct((P,), jnp.float32)),
        mesh=_mesh(),
        scratch_types=[
            pltpu.VMEM((_DSEG,), jnp.int32),
            pltpu.VMEM((_DSEG,), jnp.int32),
            pltpu.VMEM((_DSEG,), jnp.float32),
            pltpu.VMEM((_DSEG,), jnp.float32),
            pltpu.VMEM((_DSEG, H), jnp.float32),
            pltpu.SemaphoreType.DMA,
            pltpu.SemaphoreType.DMA,
            pltpu.SemaphoreType.DMA,
            pltpu.SemaphoreType.DMA,
            pltpu.SemaphoreType.DMA,
        ],
    )
    return f(d0, d1, w0, w1, x)


_DEPTH = 6


def _ffn_body(chg_ref, fid_ref, erun_ref, rr_ref, ntl_ref,
              x_ref, wg_hbm, wu_hbm, wd_hbm, ws_ref, y_ref,
              wg_b, wu_b, wd_b, sg, su, sd):
    m = pl.program_id(0)
    nrun = rr_ref[0]
    fidm = fid_ref[m]
    slot = lax.rem(fidm, _DEPTH)

    def issue(r):
        s = lax.rem(r, _DEPTH)
        e = erun_ref[r]
        pltpu.make_async_copy(wg_hbm.at[e], wg_b.at[s], sg.at[s]).start()
        pltpu.make_async_copy(wu_hbm.at[e], wu_b.at[s], su.at[s]).start()
        pltpu.make_async_copy(wd_hbm.at[e], wd_b.at[s], sd.at[s]).start()

    @pl.when(m == 0)
    def _():
        for d in range(_DEPTH):
            @pl.when(d < nrun)
            def _():
                issue(jnp.int32(d))

    @pl.when((m > 0) & (chg_ref[m] == 1))
    def _():
        r = fidm + (_DEPTH - 1)

        @pl.when(r < nrun)
        def _():
            issue(r)

    @pl.when(chg_ref[m] == 1)
    def _():
        pltpu.make_async_copy(wg_hbm.at[0], wg_b.at[slot], sg.at[slot]).wait()
        pltpu.make_async_copy(wu_hbm.at[0], wu_b.at[slot], su.at[slot]).wait()
        pltpu.make_async_copy(wd_hbm.at[0], wd_b.at[slot], sd.at[slot]).wait()

    @pl.when(m < ntl_ref[0])
    def _():
        xb = x_ref[...].astype(jnp.bfloat16)
        wg = wg_b[slot].astype(jnp.bfloat16)
        wu = wu_b[slot].astype(jnp.bfloat16)
        g = lax.dot_general(xb, wg, (((1,), (1,)), ((), ())),
                            preferred_element_type=jnp.float32)
        u = lax.dot_general(xb, wu, (((1,), (1,)), ((), ())),
                            preferred_element_type=jnp.float32)
        h = g * (1.0 / (1.0 + jnp.exp(-g))) * u
        hb = h.astype(jnp.bfloat16)
        wd = wd_b[slot].astype(jnp.bfloat16)
        y = lax.dot_general(hb, wd, (((1,), (1,)), ((), ())),
                            preferred_element_type=jnp.float32)
        y_ref[...] = y * ws_ref[...]


def _ffn(chg, fid, erun, rr, ntl, x_pad, w_gate, w_up, w_down, w_scale):
    def _xmap(m, chg, fid, erun, rr, ntl):
        return (jnp.minimum(m, ntl[0] - 1), 0)

    grid_spec = pltpu.PrefetchScalarGridSpec(
        num_scalar_prefetch=5,
        grid=(MAXT,),
        in_specs=[
            pl.BlockSpec((BT, H), _xmap),
            pl.BlockSpec(memory_space=pl.ANY),
            pl.BlockSpec(memory_space=pl.ANY),
            pl.BlockSpec(memory_space=pl.ANY),
            pl.BlockSpec((BT, 1), _xmap),
        ],
        out_specs=pl.BlockSpec((BT, H), lambda m, *_: (m, 0)),
        scratch_shapes=[
            pltpu.VMEM((_DEPTH, I, H), jnp.float32),
            pltpu.VMEM((_DEPTH, I, H), jnp.float32),
            pltpu.VMEM((_DEPTH, H, I), jnp.float32),
            pltpu.SemaphoreType.DMA((_DEPTH,)),
            pltpu.SemaphoreType.DMA((_DEPTH,)),
            pltpu.SemaphoreType.DMA((_DEPTH,)),
        ],
    )
    return pl.pallas_call(
        _ffn_body,
        grid_spec=grid_spec,
        out_shape=jax.ShapeDtypeStruct((P, H), jnp.float32),
    )(chg, fid, erun, rr, ntl, x_pad, w_gate, w_up, w_down, w_scale)


_CSEG = T // NW


def _combine(d0_hbm, d1_hbm, y_hbm, out_hbm, i0_v, i1_v, a_v, b_v, sem,
             sem2):
    wid = lax.axis_index("s") * NC + lax.axis_index("c")
    base = wid * _CSEG
    l0 = pltpu.async_copy(d0_hbm.at[pl.ds(base, _CSEG)], i0_v, sem)
    l1 = pltpu.async_copy(d1_hbm.at[pl.ds(base, _CSEG)], i1_v, sem2)
    l0.wait()
    g0 = pltpu.async_copy(y_hbm.at[i0_v], a_v, sem)
    l1.wait()
    g1 = pltpu.async_copy(y_hbm.at[i1_v], b_v, sem2)
    g0.wait()
    g1.wait()

    def row(r, _):
        for j in range(H // 16):
            s = pl.ds(j * 16, 16)
            a_v[r, s] = a_v[r, s] + b_v[r, s]
        return _

    lax.fori_loop(0, _CSEG, row, 0)
    pltpu.sync_copy(a_v, out_hbm.at[pl.ds(base, _CSEG)])


def _combine_call(d0, d1, y_pad):
    f = pl.kernel(
        _combine,
        out_type=jax.ShapeDtypeStruct((T, H), jnp.float32),
        mesh=_mesh(),
        scratch_types=[
            pltpu.VMEM((_CSEG,), jnp.int32),
            pltpu.VMEM((_CSEG,), jnp.int32),
            pltpu.VMEM((_CSEG, H), jnp.float32),
            pltpu.VMEM((_CSEG, H), jnp.float32),
            pltpu.SemaphoreType.DMA,
            pltpu.SemaphoreType.DMA,
        ],
    )
    return f(d0, d1, y_pad)


def kernel(hidden_states, gate_w, w_gate, w_up, w_down):
    d0, d1, w0, w1, chg, fid, erun, rr, ntl = _route(hidden_states, gate_w)
    x_pad, w_pad = _dispatch(d0, d1, w0, w1, hidden_states)
    y_pad = _ffn(chg, fid, erun, rr, ntl, x_pad, w_gate, w_up, w_down,
                 w_pad.reshape(P, 1))
    return _combine_call(d0, d1, y_pad)
